# baseline (speedup 1.0000x reference)
import functools
import jax
import jax.numpy as jnp
from jax import lax
from jax.experimental import pallas as pl
from jax.experimental.pallas import tpu as pltpu
from jax.experimental.pallas import tpu_sc as plsc

N = 10000
D = 128
E = 320000
CHUNK = 128
NC = 2
NS = 16
NW = NC * NS
C_PER_W = 80
E_PAD = NW * C_PER_W * CHUNK
PAD_N = 10240
ROWS_PER_TILE = PAD_N // NS
E_W = E // NW
PAD_W = C_PER_W * CHUNK - E_W
NDUM = PAD_N - N
N_COPIES = 16

_mesh = plsc.VectorSubcoreMesh(core_axis_name="c", subcore_axis_name="s")

K = 2
SG = 16
N_SG = C_PER_W // SG
G_PER_SG = SG // K


@functools.partial(
    pl.kernel,
    mesh=_mesh,
    out_type=jax.ShapeDtypeStruct((NC, PAD_N, D), jnp.float32),
    scratch_types=[
        pltpu.VMEM((SG, CHUNK), jnp.int32),
        pltpu.VMEM((SG, CHUNK), jnp.int32),
        [pltpu.VMEM((CHUNK, D), jnp.float32) for _ in range(K)],
        pltpu.VMEM_SHARED((PAD_N, D), jnp.float32),
        pltpu.SemaphoreType.DMA,
        pltpu.SemaphoreType.DMA,
    ],
)
def _sc_segment_sum(h_hbm, src_hbm, dst_hbm, zeros_hbm, out_hbm,
                    sidx_v, didx_v, rows_v, acc_sh, gsem, ssem):
    c = lax.axis_index("c")
    s = lax.axis_index("s")
    wid = s * NC + c
    pltpu.sync_copy(zeros_hbm, acc_sh.at[pl.ds(s * ROWS_PER_TILE, ROWS_PER_TILE)])
    plsc.subcore_barrier()

    def outer(sg, carry):
        pltpu.sync_copy(src_hbm.at[wid, pl.ds(sg * SG, SG)], sidx_v)
        pltpu.sync_copy(dst_hbm.at[wid, pl.ds(sg * SG, SG)], didx_v)

        def body(g, carry2):
            gh = [pltpu.async_copy(h_hbm.at[sidx_v.at[g * K + t]],
                                   rows_v[t], gsem)
                  for t in range(K)]
            sh = []
            for t in range(K):
                gh[t].wait()
                sh.append(pltpu.async_copy(
                    rows_v[t], acc_sh.at[didx_v.at[g * K + t]], ssem,
                    add=True))
            for t in range(K):
                sh[t].wait()
            return carry2

        lax.fori_loop(0, G_PER_SG, body, 0)
        return carry

    lax.fori_loop(0, N_SG, outer, 0)
    plsc.subcore_barrier()
    pltpu.sync_copy(acc_sh.at[pl.ds(s * ROWS_PER_TILE, ROWS_PER_TILE)],
                    out_hbm.at[c, pl.ds(s * ROWS_PER_TILE, ROWS_PER_TILE)])


def _tc_body(h_ref, a0_ref, a1_ref, wm_ref, w1_ref, w2_ref, b_ref, o_ref):
    dn = (((1,), (1,)), ((), ()))
    acc = a0_ref[...] + a1_ref[...]
    agg = lax.dot_general(acc, wm_ref[...], dn, preferred_element_type=jnp.float32)
    z = lax.dot_general(h_ref[...], w1_ref[...], dn, preferred_element_type=jnp.float32)
    z = z + lax.dot_general(agg, w2_ref[...], dn, preferred_element_type=jnp.float32)
    o_ref[...] = jnp.maximum(z + b_ref[...], 0.0)


def _tc_update(h, acc0, acc1, W_msg, W1, W2, b2d):
    BLK = 1000
    grid = (N // BLK,)
    row_spec = pl.BlockSpec((BLK, D), lambda i: (i, 0))
    w_spec = pl.BlockSpec((D, D), lambda i: (0, 0))
    return pl.pallas_call(
        _tc_body,
        grid=grid,
        in_specs=[row_spec, row_spec, row_spec, w_spec, w_spec, w_spec,
                  pl.BlockSpec((1, D), lambda i: (0, 0))],
        out_specs=row_spec,
        out_shape=jax.ShapeDtypeStruct((N, D), jnp.float32),
    )(h, acc0, acc1, W_msg, W1, W2, b2d)


def kernel(h, edge_index, W_msg, W_upd, b_upd):
    src = edge_index[0].astype(jnp.int32)
    dst = edge_index[1].astype(jnp.int32)
    core_off = (jnp.arange(NW, dtype=jnp.int32)[:, None] % N_COPIES) * N
    src_p = (jnp.pad(src.reshape(NW, E_W), ((0, 0), (0, PAD_W)))
             + core_off).reshape(NW, C_PER_W, CHUNK)
    w_ids = jax.lax.broadcasted_iota(jnp.int32, (NW, PAD_W), 0)
    p_ids = jax.lax.broadcasted_iota(jnp.int32, (NW, PAD_W), 1)
    pad_dst = N + (w_ids + p_ids) % NDUM
    dst_p = jnp.concatenate([dst.reshape(NW, E_W), pad_dst], axis=1).reshape(
        NW, C_PER_W, CHUNK)
    zeros = jnp.zeros((ROWS_PER_TILE, D), jnp.float32)
    h2 = jnp.concatenate([h] * N_COPIES, axis=0)
    partial = _sc_segment_sum(h2, src_p, dst_p, zeros)
    acc0 = partial[0, :N]
    acc1 = partial[1, :N]
    W1 = W_upd[:, :D]
    W2 = W_upd[:, D:]
    return _tc_update(h, acc0, acc1, W_msg, W1, W2, b_upd.reshape(1, D))


# speedup vs baseline: 1.1099x; 1.1099x over previous
import functools
import jax
import jax.numpy as jnp
from jax import lax
from jax.experimental import pallas as pl
from jax.experimental.pallas import tpu as pltpu
from jax.experimental.pallas import tpu_sc as plsc

N = 10000
D = 128
E = 320000
CHUNK = 128
NC = 2
NS = 16
NW = NC * NS
C_PER_W = 80
E_PAD = NW * C_PER_W * CHUNK
PAD_N = 10240
ROWS_PER_TILE = PAD_N // NS
E_W = E // NW
PAD_W = C_PER_W * CHUNK - E_W
NDUM = PAD_N - N
N_COPIES = 8

_mesh = plsc.VectorSubcoreMesh(core_axis_name="c", subcore_axis_name="s")

K = 2
SG = 16
N_SG = C_PER_W // SG
G_PER_SG = SG // K


@functools.partial(
    pl.kernel,
    mesh=_mesh,
    out_type=jax.ShapeDtypeStruct((NC, PAD_N, D), jnp.float32),
    scratch_types=[
        pltpu.VMEM((SG, CHUNK), jnp.int32),
        pltpu.VMEM((SG, CHUNK), jnp.int32),
        [pltpu.VMEM((CHUNK, D), jnp.float32) for _ in range(K)],
        pltpu.VMEM_SHARED((PAD_N, D), jnp.float32),
        pltpu.SemaphoreType.DMA,
        pltpu.SemaphoreType.DMA,
    ],
)
def _sc_segment_sum(h_hbm, src_hbm, dst_hbm, zeros_hbm, out_hbm,
                    sidx_v, didx_v, rows_v, acc_sh, gsem, ssem):
    c = lax.axis_index("c")
    s = lax.axis_index("s")
    wid = s * NC + c
    pltpu.sync_copy(zeros_hbm, acc_sh.at[pl.ds(s * ROWS_PER_TILE, ROWS_PER_TILE)])
    plsc.subcore_barrier()

    def outer(sg, carry):
        pltpu.sync_copy(src_hbm.at[wid, pl.ds(sg * SG, SG)], sidx_v)
        pltpu.sync_copy(dst_hbm.at[wid, pl.ds(sg * SG, SG)], didx_v)

        def body(g, carry2):
            gh = [pltpu.async_copy(h_hbm.at[sidx_v.at[g * K + t]],
                                   rows_v[t], gsem)
                  for t in range(K)]
            sh = []
            for t in range(K):
                gh[t].wait()
                sh.append(pltpu.async_copy(
                    rows_v[t], acc_sh.at[didx_v.at[g * K + t]], ssem,
                    add=True))
            for t in range(K):
                sh[t].wait()
            return carry2

        lax.fori_loop(0, G_PER_SG, body, 0)
        return carry

    lax.fori_loop(0, N_SG, outer, 0)
    plsc.subcore_barrier()
    pltpu.sync_copy(acc_sh.at[pl.ds(s * ROWS_PER_TILE, ROWS_PER_TILE)],
                    out_hbm.at[c, pl.ds(s * ROWS_PER_TILE, ROWS_PER_TILE)])


def _tc_body(h_ref, a0_ref, a1_ref, wm_ref, w1_ref, w2_ref, b_ref, o_ref):
    dn = (((1,), (1,)), ((), ()))
    acc = a0_ref[...] + a1_ref[...]
    agg = lax.dot_general(acc, wm_ref[...], dn, preferred_element_type=jnp.float32)
    z = lax.dot_general(h_ref[...], w1_ref[...], dn, preferred_element_type=jnp.float32)
    z = z + lax.dot_general(agg, w2_ref[...], dn, preferred_element_type=jnp.float32)
    o_ref[...] = jnp.maximum(z + b_ref[...], 0.0)


def _tc_update(h, acc0, acc1, W_msg, W1, W2, b2d):
    BLK = 1000
    grid = (N // BLK,)
    row_spec = pl.BlockSpec((BLK, D), lambda i: (i, 0))
    w_spec = pl.BlockSpec((D, D), lambda i: (0, 0))
    return pl.pallas_call(
        _tc_body,
        grid=grid,
        in_specs=[row_spec, row_spec, row_spec, w_spec, w_spec, w_spec,
                  pl.BlockSpec((1, D), lambda i: (0, 0))],
        out_specs=row_spec,
        out_shape=jax.ShapeDtypeStruct((N, D), jnp.float32),
    )(h, acc0, acc1, W_msg, W1, W2, b2d)


def kernel(h, edge_index, W_msg, W_upd, b_upd):
    src = edge_index[0].astype(jnp.int32)
    dst = edge_index[1].astype(jnp.int32)
    core_off = (jnp.arange(NW, dtype=jnp.int32)[:, None] % N_COPIES) * N
    src_p = (jnp.pad(src.reshape(NW, E_W), ((0, 0), (0, PAD_W)))
             + core_off).reshape(NW, C_PER_W, CHUNK)
    w_ids = jax.lax.broadcasted_iota(jnp.int32, (NW, PAD_W), 0)
    p_ids = jax.lax.broadcasted_iota(jnp.int32, (NW, PAD_W), 1)
    pad_dst = N + (w_ids + p_ids) % NDUM
    dst_p = jnp.concatenate([dst.reshape(NW, E_W), pad_dst], axis=1).reshape(
        NW, C_PER_W, CHUNK)
    zeros = jnp.zeros((ROWS_PER_TILE, D), jnp.float32)
    h2 = jnp.concatenate([h] * N_COPIES, axis=0)
    partial = _sc_segment_sum(h2, src_p, dst_p, zeros)
    acc0 = partial[0, :N]
    acc1 = partial[1, :N]
    W1 = W_upd[:, :D]
    W2 = W_upd[:, D:]
    return _tc_update(h, acc0, acc1, W_msg, W1, W2, b_upd.reshape(1, D))
